# initial kernel scaffold (unmeasured)
import jax
import jax.numpy as jnp
from jax import lax
from jax.experimental import pallas as pl
from jax.experimental.pallas import tpu as pltpu


def kernel(
    x,
):
    def body(*refs):
        pass

    out_shape = jax.ShapeDtypeStruct(..., jnp.float32)
    return pl.pallas_call(body, out_shape=out_shape)(...)


# baseline (device time: 151787 ns/iter reference)
import jax
import jax.numpy as jnp
from jax import lax
from jax.experimental import pallas as pl
from jax.experimental.pallas import tpu as pltpu

N_Z = 4


def kernel(x):
    m_per, n = x.shape

    def body(x_ref, out_ref, send_sems, recv_sems):
        my_x = lax.axis_index("x")
        my_y = lax.axis_index("y")
        my_z = lax.axis_index("z")
        right = jnp.mod(my_z + 1, N_Z)
        left = jnp.mod(my_z - 1, N_Z)

        barrier_sem = pltpu.get_barrier_semaphore()
        for nbr in (left, right):
            pl.semaphore_signal(
                barrier_sem, inc=1,
                device_id=(my_x, my_y, nbr),
                device_id_type=pl.DeviceIdType.MESH,
            )
        pl.semaphore_wait(barrier_sem, 2)

        out_ref[pl.ds(my_z * m_per, m_per), :] = x_ref[:, :]

        for h in range(N_Z - 1):
            origin = jnp.mod(my_z - h, N_Z)
            rdma = pltpu.make_async_remote_copy(
                src_ref=out_ref.at[pl.ds(origin * m_per, m_per), :],
                dst_ref=out_ref.at[pl.ds(origin * m_per, m_per), :],
                send_sem=send_sems.at[h],
                recv_sem=recv_sems.at[h],
                device_id=(my_x, my_y, right),
                device_id_type=pl.DeviceIdType.MESH,
            )
            rdma.start()
            rdma.wait()

    out_shape = jax.ShapeDtypeStruct((N_Z * m_per, n), x.dtype)
    return pl.pallas_call(
        body,
        out_shape=out_shape,
        in_specs=[pl.BlockSpec(memory_space=pltpu.VMEM)],
        out_specs=pl.BlockSpec(memory_space=pltpu.VMEM),
        scratch_shapes=[
            pltpu.SemaphoreType.DMA((N_Z - 1,)),
            pltpu.SemaphoreType.DMA((N_Z - 1,)),
        ],
        compiler_params=pltpu.CompilerParams(collective_id=0),
    )(x)


# device time: 83222 ns/iter; 1.8239x vs baseline; 1.8239x over previous
import jax
import jax.numpy as jnp
from jax import lax
from jax.experimental import pallas as pl
from jax.experimental.pallas import tpu as pltpu

N_Z = 4
N_HOP = N_Z - 1


def kernel(x):
    m_per, n = x.shape
    qr = m_per // 4
    hr = qr // 2

    def body(x_ref, out_ref,
             ring_s, ring_r, sx_s, sx_r, sy_s, sy_r,
             rly_s, rly_r, rlx_s, rlx_r):
        my_x = lax.axis_index("x")
        my_y = lax.axis_index("y")
        my_z = lax.axis_index("z")
        ox = 1 - my_x
        oy = 1 - my_y
        zr = jnp.mod(my_z + 1, N_Z)
        zl = jnp.mod(my_z - 1, N_Z)
        q = 2 * my_x + my_y
        qx = 2 * ox + my_y
        qy = 2 * my_x + oy

        barrier_sem = pltpu.get_barrier_semaphore()
        for dev in ((my_x, my_y, zl), (my_x, my_y, zr),
                    (ox, my_y, my_z), (my_x, oy, my_z)):
            pl.semaphore_signal(barrier_sem, inc=1, device_id=dev,
                                device_id_type=pl.DeviceIdType.MESH)
        pl.semaphore_wait(barrier_sem, 4)

        def quarter(c, qi):
            return out_ref.at[pl.ds(c * m_per + qi * qr, qr), :]

        all_rdmas = []

        def ring_hop(h):
            cs = jnp.mod(my_z - h, N_Z)
            src = (x_ref.at[pl.ds(q * qr, qr), :] if h == 0
                   else quarter(cs, q))
            r = pltpu.make_async_remote_copy(
                src_ref=src, dst_ref=quarter(cs, q),
                send_sem=ring_s.at[h], recv_sem=ring_r.at[h],
                device_id=(my_x, my_y, zr),
                device_id_type=pl.DeviceIdType.MESH)
            all_rdmas.append(r)
            return r

        ring = [None] * N_HOP
        ring[0] = ring_hop(0)
        ring[0].start()

        out_ref[pl.ds(my_z * m_per, m_per), :] = x_ref[:, :]

        s4 = [None] * N_HOP
        s5 = [None] * N_HOP
        for h in range(N_HOP):
            cr = jnp.mod(my_z - 1 - h, N_Z)
            ring[h].wait_recv()
            if h + 1 < N_HOP:
                ring[h + 1] = ring_hop(h + 1)
                ring[h + 1].start()
            sx = pltpu.make_async_remote_copy(
                src_ref=quarter(cr, q), dst_ref=quarter(cr, q),
                send_sem=sx_s.at[h], recv_sem=sx_r.at[h],
                device_id=(ox, my_y, my_z),
                device_id_type=pl.DeviceIdType.MESH)
            sy = pltpu.make_async_remote_copy(
                src_ref=quarter(cr, q), dst_ref=quarter(cr, q),
                send_sem=sy_s.at[h], recv_sem=sy_r.at[h],
                device_id=(my_x, oy, my_z),
                device_id_type=pl.DeviceIdType.MESH)
            all_rdmas += [sx, sy]
            sx.start()
            sy.start()
            sx.wait_recv()
            s4[h] = pltpu.make_async_remote_copy(
                src_ref=out_ref.at[pl.ds(cr * m_per + qx * qr + hr, hr), :],
                dst_ref=out_ref.at[pl.ds(cr * m_per + qx * qr + hr, hr), :],
                send_sem=rly_s.at[h], recv_sem=rly_r.at[h],
                device_id=(my_x, oy, my_z),
                device_id_type=pl.DeviceIdType.MESH)
            all_rdmas.append(s4[h])
            s4[h].start()
            sy.wait_recv()
            s5[h] = pltpu.make_async_remote_copy(
                src_ref=out_ref.at[pl.ds(cr * m_per + qy * qr, hr), :],
                dst_ref=out_ref.at[pl.ds(cr * m_per + qy * qr, hr), :],
                send_sem=rlx_s.at[h], recv_sem=rlx_r.at[h],
                device_id=(ox, my_y, my_z),
                device_id_type=pl.DeviceIdType.MESH)
            all_rdmas.append(s5[h])
            s5[h].start()

        for h in range(N_HOP):
            s4[h].wait_recv()
            s5[h].wait_recv()

        for r in all_rdmas:
            r.wait_send()

    out_shape = jax.ShapeDtypeStruct((N_Z * m_per, n), x.dtype)
    sem = pltpu.SemaphoreType.DMA((N_HOP,))
    return pl.pallas_call(
        body,
        out_shape=out_shape,
        in_specs=[pl.BlockSpec(memory_space=pltpu.VMEM)],
        out_specs=pl.BlockSpec(memory_space=pltpu.VMEM),
        scratch_shapes=[sem] * 10,
        compiler_params=pltpu.CompilerParams(collective_id=0),
    )(x)
